# Initial kernel scaffold; baseline (speedup 1.0000x reference)
import jax
import jax.numpy as jnp
from jax.experimental import pallas as pl


def kernel(features, edge_index, batch_nodes, W1_self, W1_neigh, b1, W2_self, W2_neigh, b2, W_lin, b_lin):
    raise NotImplementedError("write your pallas kernel here")


# speedup vs baseline: 4.0690x; 4.0690x over previous
import functools

import jax
import jax.numpy as jnp
from jax import lax
from jax.experimental import pallas as pl
from jax.experimental.pallas import tpu as pltpu
from jax.experimental.pallas import tpu_sc as plsc

NC = 2
NS = 16
LANES = 128


def _sc_agg_body(x_hbm, src2d, dst2d, agg_out,
                 agg_sh, srci, dsti, rows, sem):
    cid = lax.axis_index("c")
    sid = lax.axis_index("s")
    wid = cid * NS + sid
    np_rows = agg_sh.shape[0]
    stripe = np_rows // NS
    nblk = stripe // LANES
    chunks = src2d.shape[0] // (NC * NS)

    z16 = jnp.zeros((16,), jnp.float32)
    d = rows.shape[1]

    def zero_bufs(i, c):
        for j in range(d // 16):
            rows[i, pl.ds(j * 16, 16)] = z16
        return c

    lax.fori_loop(0, LANES, zero_bufs, 0)

    def zero_stripe(i, c):
        off = sid * stripe + i * LANES
        pltpu.sync_copy(rows, agg_sh.at[pl.ds(off, LANES)])
        return c

    lax.fori_loop(0, nblk, zero_stripe, 0)
    plsc.subcore_barrier()

    def body(k, c):
        r = wid * chunks + k
        pltpu.sync_copy(src2d.at[r], srci)
        pltpu.sync_copy(dst2d.at[r], dsti)
        pltpu.async_copy(x_hbm.at[srci], rows, sem).wait()
        pltpu.sync_copy(rows, agg_sh.at[dsti], add=True)
        return c

    lax.fori_loop(0, chunks, body, 0)
    plsc.subcore_barrier()

    def writeout(i, c):
        off = sid * stripe + i * LANES
        pltpu.sync_copy(agg_sh.at[pl.ds(off, LANES)], rows)
        pltpu.sync_copy(rows, agg_out.at[pl.ds(cid * np_rows + off, LANES)])
        return c

    lax.fori_loop(0, nblk, writeout, 0)


def _sc_agg(x, src2d, dst2d, np_rows):
    d = x.shape[1]
    f = pl.kernel(
        _sc_agg_body,
        out_type=jax.ShapeDtypeStruct((NC * np_rows, d), jnp.float32),
        mesh=plsc.VectorSubcoreMesh(core_axis_name="c", subcore_axis_name="s"),
        scratch_types=[
            pltpu.VMEM_SHARED((np_rows, d), jnp.float32),
            pltpu.VMEM((LANES,), jnp.int32),
            pltpu.VMEM((LANES,), jnp.int32),
            pltpu.VMEM((LANES, d), jnp.float32),
            pltpu.SemaphoreType.DMA,
        ],
    )
    return f(x, src2d, dst2d).reshape(NC, np_rows, d)


def _sc_deg_body(dst2d, deg_out, deg_sh, dsti, onesb, sem):
    cid = lax.axis_index("c")
    sid = lax.axis_index("s")
    wid = cid * NS + sid
    np_rows = deg_sh.shape[0]
    stripe = np_rows // NS
    nblk = stripe // LANES
    chunks = dst2d.shape[0] // (NC * NS)

    z16 = jnp.zeros((16,), jnp.float32)
    ones16 = jnp.ones((16,), jnp.float32)

    def zero_bufs(i, c):
        for j in range(LANES // 16):
            onesb[i, pl.ds(j * 16, 16)] = z16
        return c

    lax.fori_loop(0, LANES, zero_bufs, 0)

    def zero_stripe(i, c):
        off = sid * stripe + i * LANES
        pltpu.sync_copy(onesb, deg_sh.at[pl.ds(off, LANES)])
        return c

    lax.fori_loop(0, nblk, zero_stripe, 0)

    def fill_ones(i, c):
        for j in range(LANES // 16):
            onesb[i, pl.ds(j * 16, 16)] = ones16
        return c

    lax.fori_loop(0, LANES, fill_ones, 0)
    plsc.subcore_barrier()

    def body(k, c):
        r = wid * chunks + k
        pltpu.sync_copy(dst2d.at[r], dsti)
        pltpu.sync_copy(onesb, deg_sh.at[dsti], add=True)
        return c

    lax.fori_loop(0, chunks, body, 0)
    plsc.subcore_barrier()

    def writeout(i, c):
        off = sid * stripe + i * LANES
        pltpu.sync_copy(deg_sh.at[pl.ds(off, LANES)], onesb)
        pltpu.sync_copy(onesb, deg_out.at[pl.ds(cid * np_rows + off, LANES)])
        return c

    lax.fori_loop(0, nblk, writeout, 0)


def _sc_deg(dst2d, np_rows):
    f = pl.kernel(
        _sc_deg_body,
        out_type=jax.ShapeDtypeStruct((NC * np_rows, LANES), jnp.float32),
        mesh=plsc.VectorSubcoreMesh(core_axis_name="c", subcore_axis_name="s"),
        scratch_types=[
            pltpu.VMEM_SHARED((np_rows, LANES), jnp.float32),
            pltpu.VMEM((LANES,), jnp.int32),
            pltpu.VMEM((LANES, LANES), jnp.float32),
            pltpu.SemaphoreType.DMA,
        ],
    )
    return f(dst2d).reshape(NC, np_rows, LANES)


def _sc_gather_body(h_hbm, idx_hbm, out_hbm, idxv, rowsv, sem):
    cid = lax.axis_index("c")
    sid = lax.axis_index("s")
    wid = cid * NS + sid
    bpw = idxv.shape[0]
    base = wid * bpw
    pltpu.sync_copy(idx_hbm.at[pl.ds(base, bpw)], idxv)
    pltpu.async_copy(h_hbm.at[idxv], rowsv, sem).wait()
    pltpu.sync_copy(rowsv, out_hbm.at[pl.ds(base, bpw)])


def _sc_gather(h, idx):
    b = idx.shape[0]
    d = h.shape[1]
    bpw = b // (NC * NS)
    f = pl.kernel(
        _sc_gather_body,
        out_type=jax.ShapeDtypeStruct((b, d), jnp.float32),
        mesh=plsc.VectorSubcoreMesh(core_axis_name="c", subcore_axis_name="s"),
        scratch_types=[
            pltpu.VMEM((bpw,), jnp.int32),
            pltpu.VMEM((bpw, d), jnp.float32),
            pltpu.SemaphoreType.DMA,
        ],
    )
    return f(h, idx)


def _tc_layer_body(x_ref, a0_ref, a1_ref, d0_ref, d1_ref, ws_ref, wn_ref,
                   b_ref, o_ref):
    x = x_ref[...]
    a = a0_ref[0] + a1_ref[0]
    deg = (d0_ref[0] + d1_ref[0])[:, :1]
    mean = a * (1.0 / jnp.clip(deg, 1.0, None))
    h = jnp.dot(x, ws_ref[...], preferred_element_type=jnp.float32)
    h = h + jnp.dot(mean, wn_ref[...], preferred_element_type=jnp.float32)
    o_ref[...] = jnp.maximum(h + b_ref[...], 0.0)


def _tc_layer(x, aggp, degp, w_self, w_neigh, b):
    n, d = x.shape
    hh = w_self.shape[1]
    br = 512
    grid = (pl.cdiv(n, br),)
    return pl.pallas_call(
        _tc_layer_body,
        grid=grid,
        in_specs=[
            pl.BlockSpec((br, d), lambda i: (i, 0)),
            pl.BlockSpec((1, br, d), lambda i: (0, i, 0)),
            pl.BlockSpec((1, br, d), lambda i: (1, i, 0)),
            pl.BlockSpec((1, br, LANES), lambda i: (0, i, 0)),
            pl.BlockSpec((1, br, LANES), lambda i: (1, i, 0)),
            pl.BlockSpec((hh, hh), lambda i: (0, 0)),
            pl.BlockSpec((hh, hh), lambda i: (0, 0)),
            pl.BlockSpec((1, hh), lambda i: (0, 0)),
        ],
        out_specs=pl.BlockSpec((br, hh), lambda i: (i, 0)),
        out_shape=jax.ShapeDtypeStruct((n, hh), jnp.float32),
    )(x, aggp, aggp, degp, degp, w_self, w_neigh, b.reshape(1, hh))


def _tc_head_body(h_ref, w_ref, b_ref, o_ref):
    logits = jnp.dot(h_ref[...], w_ref[...],
                     preferred_element_type=jnp.float32) + b_ref[...]
    m = jnp.max(logits, axis=1, keepdims=True)
    lse = jnp.log(jnp.sum(jnp.exp(logits - m), axis=1, keepdims=True)) + m
    o_ref[...] = logits - lse


def _tc_head(hb, w_lin, b_lin):
    b, _ = hb.shape
    c = w_lin.shape[1]
    return pl.pallas_call(
        _tc_head_body,
        out_shape=jax.ShapeDtypeStruct((b, c), jnp.float32),
    )(hb, w_lin, b_lin.reshape(1, c))


def kernel(features, edge_index, batch_nodes, W1_self, W1_neigh, b1,
           W2_self, W2_neigh, b2, W_lin, b_lin):
    n, d = features.shape
    e = edge_index.shape[1]

    ce = NC * NS * LANES
    ep = ((e + ce - 1) // ce) * ce
    np_rows = ((n + 1 + 255) // 256) * 256

    src = jnp.concatenate(
        [edge_index[0], jnp.zeros((ep - e,), jnp.int32)]).reshape(-1, LANES)
    dst = jnp.concatenate(
        [edge_index[1], jnp.full((ep - e,), n, jnp.int32)]).reshape(-1, LANES)

    degp = _sc_deg(dst, np_rows)
    agg1 = _sc_agg(features, src, dst, np_rows)
    h1 = _tc_layer(features, agg1, degp, W1_self, W1_neigh, b1)
    agg2 = _sc_agg(h1, src, dst, np_rows)
    h2 = _tc_layer(h1, agg2, degp, W2_self, W2_neigh, b2)
    hb = _sc_gather(h2, batch_nodes)
    return _tc_head(hb, W_lin, b_lin)


# speedup vs baseline: 4.3308x; 1.0643x over previous
import functools

import jax
import jax.numpy as jnp
from jax import lax
from jax.experimental import pallas as pl
from jax.experimental.pallas import tpu as pltpu
from jax.experimental.pallas import tpu_sc as plsc

NC = 2
NS = 16
LANES = 128
KE = 64


def _sc_agg_body(x_hbm, src2d, dst2d, agg_out,
                 agg_sh, srci0, dsti0, rows0, srci1, dsti1, rows1,
                 sem0, sem1):
    cid = lax.axis_index("c")
    sid = lax.axis_index("s")
    wid = cid * NS + sid
    np_rows = agg_sh.shape[0]
    stripe = np_rows // NS
    nblk = stripe // KE
    chunks = src2d.shape[0] // (NC * NS)
    npair = chunks // 2
    base = wid * chunks

    z16 = jnp.zeros((16,), jnp.float32)
    d = rows0.shape[1]

    def zero_bufs(i, c):
        for j in range(d // 16):
            rows0[i, pl.ds(j * 16, 16)] = z16
        return c

    lax.fori_loop(0, KE, zero_bufs, 0)

    def zero_stripe(i, c):
        off = sid * stripe + i * KE
        pltpu.sync_copy(rows0, agg_sh.at[pl.ds(off, KE)])
        return c

    lax.fori_loop(0, nblk, zero_stripe, 0)
    plsc.subcore_barrier()

    pltpu.sync_copy(src2d.at[base], srci0)
    pltpu.sync_copy(dst2d.at[base], dsti0)
    pltpu.async_copy(x_hbm.at[srci0], rows0, sem0)

    def pair(p, c):
        r1 = base + 2 * p + 1
        pltpu.sync_copy(src2d.at[r1], srci1)
        pltpu.sync_copy(dst2d.at[r1], dsti1)
        pltpu.async_copy(x_hbm.at[srci1], rows1, sem1)
        pltpu.make_async_copy(x_hbm.at[srci0], rows0, sem0).wait()
        pltpu.sync_copy(rows0, agg_sh.at[dsti0], add=True)

        @pl.when(p < npair - 1)
        def _():
            r0 = base + 2 * p + 2
            pltpu.sync_copy(src2d.at[r0], srci0)
            pltpu.sync_copy(dst2d.at[r0], dsti0)
            pltpu.async_copy(x_hbm.at[srci0], rows0, sem0)

        pltpu.make_async_copy(x_hbm.at[srci1], rows1, sem1).wait()
        pltpu.sync_copy(rows1, agg_sh.at[dsti1], add=True)
        return c

    lax.fori_loop(0, npair, pair, 0)
    plsc.subcore_barrier()

    def writeout(i, c):
        off = sid * stripe + i * KE
        pltpu.sync_copy(agg_sh.at[pl.ds(off, KE)], rows0)
        pltpu.sync_copy(rows0, agg_out.at[pl.ds(cid * np_rows + off, KE)])
        return c

    lax.fori_loop(0, nblk, writeout, 0)


def _sc_agg(x, src2d, dst2d, np_rows):
    d = x.shape[1]
    f = pl.kernel(
        _sc_agg_body,
        out_type=jax.ShapeDtypeStruct((NC * np_rows, d), jnp.float32),
        mesh=plsc.VectorSubcoreMesh(core_axis_name="c", subcore_axis_name="s"),
        scratch_types=[
            pltpu.VMEM_SHARED((np_rows, d), jnp.float32),
            pltpu.VMEM((KE,), jnp.int32),
            pltpu.VMEM((KE,), jnp.int32),
            pltpu.VMEM((KE, d), jnp.float32),
            pltpu.VMEM((KE,), jnp.int32),
            pltpu.VMEM((KE,), jnp.int32),
            pltpu.VMEM((KE, d), jnp.float32),
            pltpu.SemaphoreType.DMA,
            pltpu.SemaphoreType.DMA,
        ],
    )
    return f(x, src2d, dst2d).reshape(NC, np_rows, d)


def _sc_deg_body(dst2d, deg_out, deg_sh, dsti, onesb, sem):
    cid = lax.axis_index("c")
    sid = lax.axis_index("s")
    wid = cid * NS + sid
    np_rows = deg_sh.shape[0]
    stripe = np_rows // NS
    nblk = stripe // KE
    chunks = dst2d.shape[0] // (NC * NS)

    z16 = jnp.zeros((16,), jnp.float32)
    ones16 = jnp.ones((16,), jnp.float32)

    def zero_bufs(i, c):
        for j in range(LANES // 16):
            onesb[i, pl.ds(j * 16, 16)] = z16
        return c

    lax.fori_loop(0, KE, zero_bufs, 0)

    def zero_stripe(i, c):
        off = sid * stripe + i * KE
        pltpu.sync_copy(onesb, deg_sh.at[pl.ds(off, KE)])
        return c

    lax.fori_loop(0, nblk, zero_stripe, 0)

    def fill_ones(i, c):
        for j in range(LANES // 16):
            onesb[i, pl.ds(j * 16, 16)] = ones16
        return c

    lax.fori_loop(0, KE, fill_ones, 0)
    plsc.subcore_barrier()

    def body(k, c):
        r = wid * chunks + k
        pltpu.sync_copy(dst2d.at[r], dsti)
        pltpu.sync_copy(onesb, deg_sh.at[dsti], add=True)
        return c

    lax.fori_loop(0, chunks, body, 0)
    plsc.subcore_barrier()

    def writeout(i, c):
        off = sid * stripe + i * KE
        pltpu.sync_copy(deg_sh.at[pl.ds(off, KE)], onesb)
        pltpu.sync_copy(onesb, deg_out.at[pl.ds(cid * np_rows + off, KE)])
        return c

    lax.fori_loop(0, nblk, writeout, 0)


def _sc_deg(dst2d, np_rows):
    f = pl.kernel(
        _sc_deg_body,
        out_type=jax.ShapeDtypeStruct((NC * np_rows, LANES), jnp.float32),
        mesh=plsc.VectorSubcoreMesh(core_axis_name="c", subcore_axis_name="s"),
        scratch_types=[
            pltpu.VMEM_SHARED((np_rows, LANES), jnp.float32),
            pltpu.VMEM((KE,), jnp.int32),
            pltpu.VMEM((KE, LANES), jnp.float32),
            pltpu.SemaphoreType.DMA,
        ],
    )
    return f(dst2d).reshape(NC, np_rows, LANES)


def _sc_gather_body(h_hbm, idx_hbm, out_hbm, idxv, rowsv, sem):
    cid = lax.axis_index("c")
    sid = lax.axis_index("s")
    wid = cid * NS + sid
    bpw = idxv.shape[0]
    base = wid * bpw
    pltpu.sync_copy(idx_hbm.at[pl.ds(base, bpw)], idxv)
    pltpu.async_copy(h_hbm.at[idxv], rowsv, sem).wait()
    pltpu.sync_copy(rowsv, out_hbm.at[pl.ds(base, bpw)])


def _sc_gather(h, idx):
    b = idx.shape[0]
    d = h.shape[1]
    bpw = b // (NC * NS)
    f = pl.kernel(
        _sc_gather_body,
        out_type=jax.ShapeDtypeStruct((b, d), jnp.float32),
        mesh=plsc.VectorSubcoreMesh(core_axis_name="c", subcore_axis_name="s"),
        scratch_types=[
            pltpu.VMEM((bpw,), jnp.int32),
            pltpu.VMEM((bpw, d), jnp.float32),
            pltpu.SemaphoreType.DMA,
        ],
    )
    return f(h, idx)


def _tc_layer_body(x_ref, a0_ref, a1_ref, d0_ref, d1_ref, ws_ref, wn_ref,
                   b_ref, o_ref):
    x = x_ref[...]
    a = a0_ref[0] + a1_ref[0]
    deg = (d0_ref[0] + d1_ref[0])[:, :1]
    mean = a * (1.0 / jnp.clip(deg, 1.0, None))
    h = jnp.dot(x, ws_ref[...], preferred_element_type=jnp.float32)
    h = h + jnp.dot(mean, wn_ref[...], preferred_element_type=jnp.float32)
    o_ref[...] = jnp.maximum(h + b_ref[...], 0.0)


def _tc_layer(x, aggp, degp, w_self, w_neigh, b):
    n, d = x.shape
    hh = w_self.shape[1]
    br = 512
    grid = (pl.cdiv(n, br),)
    return pl.pallas_call(
        _tc_layer_body,
        grid=grid,
        in_specs=[
            pl.BlockSpec((br, d), lambda i: (i, 0)),
            pl.BlockSpec((1, br, d), lambda i: (0, i, 0)),
            pl.BlockSpec((1, br, d), lambda i: (1, i, 0)),
            pl.BlockSpec((1, br, LANES), lambda i: (0, i, 0)),
            pl.BlockSpec((1, br, LANES), lambda i: (1, i, 0)),
            pl.BlockSpec((hh, hh), lambda i: (0, 0)),
            pl.BlockSpec((hh, hh), lambda i: (0, 0)),
            pl.BlockSpec((1, hh), lambda i: (0, 0)),
        ],
        out_specs=pl.BlockSpec((br, hh), lambda i: (i, 0)),
        out_shape=jax.ShapeDtypeStruct((n, hh), jnp.float32),
    )(x, aggp, aggp, degp, degp, w_self, w_neigh, b.reshape(1, hh))


def _tc_head_body(h_ref, w_ref, b_ref, o_ref):
    logits = jnp.dot(h_ref[...], w_ref[...],
                     preferred_element_type=jnp.float32) + b_ref[...]
    m = jnp.max(logits, axis=1, keepdims=True)
    lse = jnp.log(jnp.sum(jnp.exp(logits - m), axis=1, keepdims=True)) + m
    o_ref[...] = logits - lse


def _tc_head(hb, w_lin, b_lin):
    b, _ = hb.shape
    c = w_lin.shape[1]
    return pl.pallas_call(
        _tc_head_body,
        out_shape=jax.ShapeDtypeStruct((b, c), jnp.float32),
    )(hb, w_lin, b_lin.reshape(1, c))


def kernel(features, edge_index, batch_nodes, W1_self, W1_neigh, b1,
           W2_self, W2_neigh, b2, W_lin, b_lin):
    n, d = features.shape
    e = edge_index.shape[1]

    ce = NC * NS * LANES
    ep = ((e + ce - 1) // ce) * ce
    np_rows = ((n + 1 + 255) // 256) * 256

    src = jnp.concatenate(
        [edge_index[0], jnp.zeros((ep - e,), jnp.int32)]).reshape(-1, KE)
    dst = jnp.concatenate(
        [edge_index[1], jnp.full((ep - e,), n, jnp.int32)]).reshape(-1, KE)

    degp = _sc_deg(dst, np_rows)
    agg1 = _sc_agg(features, src, dst, np_rows)
    h1 = _tc_layer(features, agg1, degp, W1_self, W1_neigh, b1)
    agg2 = _sc_agg(h1, src, dst, np_rows)
    h2 = _tc_layer(h1, agg2, degp, W2_self, W2_neigh, b2)
    hb = _sc_gather(h2, batch_nodes)
    return _tc_head(hb, W_lin, b_lin)
